# baseline (speedup 1.0000x reference)
import functools

import jax
import jax.numpy as jnp
from jax import lax
from jax.experimental import pallas as pl
from jax.experimental.pallas import tpu as pltpu
from jax.experimental.pallas import tpu_sc as plsc

VOCAB = 1_000_000
WORLD = 4
MY_RANK = 1
PART = VOCAB // WORLD
LO = MY_RANK * PART
HI = LO + PART
EMB = 64
BATCH = 4096
SEQ = 200
NTOK = BATCH * SEQ
NPAD = 1024

NC = 2
NS = 16
NW = NC * NS
PER_W = NTOK // NW
C = 256
NCH = PER_W // C
NBUF = 4
L = 16


@functools.partial(
    pl.kernel,
    out_type=jax.ShapeDtypeStruct((NTOK, EMB), jnp.float32),
    mesh=plsc.VectorSubcoreMesh(core_axis_name="c", subcore_axis_name="s"),
    compiler_params=pltpu.CompilerParams(use_tc_tiling_on_sc=False),
    scratch_types=(
        [pltpu.VMEM((PER_W,), jnp.int32)]
        + [pltpu.VMEM((C, EMB), jnp.float32) for _ in range(NBUF)]
        + [pltpu.SemaphoreType.DMA for _ in range(2 * NBUF)]
    ),
)
def _sc_gather(ids_hbm, tab_hbm, out_hbm, sidv, *bufs):
    rows = bufs[:NBUF]
    gsem = bufs[NBUF:2 * NBUF]
    wsem = bufs[2 * NBUF:]
    wid = lax.axis_index("s") * NC + lax.axis_index("c")
    base = wid * PER_W

    pltpu.sync_copy(ids_hbm.at[pl.ds(base, PER_W)], sidv)

    def vec(i, c2):
        v = sidv[pl.ds(i * L, L)]
        m = (v >= LO) & (v < HI)
        sidv[pl.ds(i * L, L)] = jnp.where(m, v - LO, PART + (v & (NPAD - 1)))
        return c2

    lax.fori_loop(0, PER_W // L, vec, 0)

    def gather(j, b):
        pltpu.async_copy(tab_hbm.at[sidv.at[pl.ds(j * C, C)]], rows[b], gsem[b])

    def write(j, b):
        pltpu.async_copy(rows[b], out_hbm.at[pl.ds(base + j * C, C)], wsem[b])

    def wait_g(b):
        pltpu.make_async_copy(tab_hbm.at[sidv.at[pl.ds(0, C)]], rows[b], gsem[b]).wait()

    def wait_w(b):
        pltpu.make_async_copy(rows[b], out_hbm.at[pl.ds(base, C)], wsem[b]).wait()

    for b in range(NBUF):
        gather(b, b)

    def step(g, c2):
        for b in range(NBUF):
            wait_g(b)
            write(g * NBUF + b, b)
        for b in range(NBUF):
            wait_w(b)
            gather((g + 1) * NBUF + b, b)
        return c2

    lax.fori_loop(0, NCH // NBUF - 1, step, 0)

    for b in range(NBUF):
        wait_g(b)
        write((NCH // NBUF - 1) * NBUF + b, b)
    for b in range(NBUF):
        wait_w(b)


def kernel(input_ids, tr):
    ids = input_ids.reshape(NTOK)
    tab = jnp.concatenate([tr, jnp.zeros((NPAD, EMB), jnp.float32)], axis=0)
    out = _sc_gather(ids, tab)
    return out.reshape(BATCH, SEQ, EMB)


# speedup vs baseline: 1.2725x; 1.2725x over previous
import functools

import jax
import jax.numpy as jnp
from jax import lax
from jax.experimental import pallas as pl
from jax.experimental.pallas import tpu as pltpu
from jax.experimental.pallas import tpu_sc as plsc

VOCAB = 1_000_000
WORLD = 4
MY_RANK = 1
PART = VOCAB // WORLD
LO = MY_RANK * PART
HI = LO + PART
EMB = 64
BATCH = 4096
SEQ = 200
NTOK = BATCH * SEQ

NC = 2
NS = 16
NW = NC * NS
PER_W = NTOK // NW
L = 16
C = 256
CZ = 512
NZ = PER_W // CZ
G = PER_W // L


@functools.partial(
    pl.kernel,
    out_type=jax.ShapeDtypeStruct((NTOK, EMB), jnp.float32),
    mesh=plsc.VectorSubcoreMesh(core_axis_name="c", subcore_axis_name="s"),
    compiler_params=pltpu.CompilerParams(
        use_tc_tiling_on_sc=False, needs_layout_passes=False),
    scratch_types=[
        pltpu.VMEM((PER_W,), jnp.int32),
        pltpu.VMEM((PER_W,), jnp.int32),
        pltpu.VMEM((CZ, EMB), jnp.float32),
        pltpu.VMEM((C, EMB), jnp.float32),
        pltpu.VMEM((C, EMB), jnp.float32),
        pltpu.SemaphoreType.DMA,
        pltpu.SemaphoreType.DMA,
        pltpu.SemaphoreType.DMA,
        pltpu.SemaphoreType.DMA,
        pltpu.SemaphoreType.DMA,
    ],
)
def _sc_gather(ids_hbm, tab_hbm, out_hbm, idv, dposc, zbuf, r0, r1,
               zsem, gs0, gs1, ws0, ws1):
    wid = lax.axis_index("s") * NC + lax.axis_index("c")
    base = wid * PER_W

    pltpu.sync_copy(ids_hbm.at[pl.ds(base, PER_W)], idv)

    zv = jnp.zeros((L,), jnp.float32)

    def zr(r, c2):
        for k in range(EMB // L):
            zbuf[r, pl.ds(k * L, L)] = zv
        return c2

    lax.fori_loop(0, CZ, zr, 0)

    for i in range(NZ):
        pltpu.async_copy(zbuf, out_hbm.at[pl.ds(base + i * CZ, CZ)], zsem)

    ii = lax.iota(jnp.int32, L)

    def comp(i, cnt):
        v = idv[pl.ds(i * L, L)]
        m = (v >= LO) & (v < HI)
        plsc.store_compressed(idv.at[pl.ds(cnt, L)], v - LO, mask=m)
        plsc.store_compressed(dposc.at[pl.ds(cnt, L)], (base + i * L) + ii, mask=m)
        return cnt + jnp.sum(m.astype(jnp.int32))

    cnt = lax.fori_loop(0, G, comp, jnp.int32(0))

    neg = jnp.int32(-2147483648)
    s0 = jnp.max(jnp.where(ii == 0, idv[pl.ds(0, L)], neg))
    p0 = jnp.max(jnp.where(ii == 0, dposc[pl.ds(0, L)], neg))
    sidpad = jnp.full((L,), s0, jnp.int32)
    dpospad = jnp.full((L,), p0, jnp.int32)

    @pl.when((cnt & 15) != 0)
    def _():
        gg = (cnt >> 4) << 4
        keep = (gg + ii) < cnt
        idv[pl.ds(gg, L)] = jnp.where(keep, idv[pl.ds(gg, L)], sidpad)
        dposc[pl.ds(gg, L)] = jnp.where(keep, dposc[pl.ds(gg, L)], dpospad)

    nfull = (cnt + (C - 1)) >> 8
    glo = (cnt + 15) >> 4
    ghi = nfull << 4

    def padg(g, c2):
        idv[pl.ds(g * L, L)] = sidpad
        dposc[pl.ds(g * L, L)] = dpospad
        return c2

    lax.fori_loop(glo, ghi, padg, 0)

    def gat(j, rb, sb):
        pltpu.async_copy(tab_hbm.at[idv.at[pl.ds(j * C, C)]], rb, sb)

    def wat_g(rb, sb):
        pltpu.make_async_copy(tab_hbm.at[idv.at[pl.ds(0, C)]], rb, sb).wait()

    def sca(j, rb, sb):
        pltpu.async_copy(rb, out_hbm.at[dposc.at[pl.ds(j * C, C)]], sb)

    def wat_w(rb, sb):
        pltpu.make_async_copy(rb, out_hbm.at[dposc.at[pl.ds(0, C)]], sb).wait()

    @pl.when(nfull > 0)
    def _():
        gat(0, r0, gs0)

    @pl.when(nfull > 1)
    def _():
        gat(1, r1, gs1)

    for i in range(NZ):
        pltpu.make_async_copy(zbuf, out_hbm.at[pl.ds(base, CZ)], zsem).wait()

    def step(g, c2):
        j0 = 2 * g
        j1 = 2 * g + 1

        @pl.when(j0 < nfull)
        def _():
            wat_g(r0, gs0)
            sca(j0, r0, ws0)

        @pl.when(j1 < nfull)
        def _():
            wat_g(r1, gs1)
            sca(j1, r1, ws1)

        @pl.when(j0 + 2 < nfull)
        def _():
            wat_w(r0, ws0)
            gat(j0 + 2, r0, gs0)

        @pl.when(j1 + 2 < nfull)
        def _():
            wat_w(r1, ws1)
            gat(j1 + 2, r1, gs1)

        return c2

    lax.fori_loop(0, (nfull + 1) >> 1, step, 0)

    @pl.when(nfull > 0)
    def _():
        wat_w(r0, ws0)

    @pl.when(nfull > 1)
    def _():
        wat_w(r1, ws1)


def kernel(input_ids, tr):
    ids = input_ids.reshape(NTOK)
    out = _sc_gather(ids, tr)
    return out.reshape(BATCH, SEQ, EMB)
